# baseline (speedup 1.0000x reference)
import functools

import numpy as np
import jax
from jax import lax
import jax.numpy as jnp
from jax.experimental import pallas as pl
from jax.experimental.pallas import tpu as pltpu
from jax.experimental.pallas import tpu_sc as plsc

NUM_FRAMES = 16
TUBELET = 2
PATCH = 16
EMBED_DIMS = 768


def _sinusoid_table(n_position, embed_dims, base=10000):
    vec = np.arange(embed_dims, dtype=np.float64)
    vec = (vec - vec % 2) / embed_dims
    vec = np.power(float(base), -vec).reshape(1, -1)
    table = np.arange(n_position, dtype=np.float64).reshape(-1, 1) * vec
    table[:, 0::2] = np.sin(table[:, 0::2])
    table[:, 1::2] = np.cos(table[:, 1::2])
    return table.astype(np.float32)


def _body(f_ref, tok_ref, fr_ref):
    f = f_ref[0]
    fc = f.reshape(TUBELET, 224, 7, 96)
    l = jax.lax.broadcasted_iota(jnp.int32, (TUBELET, 224, 7, 96), 3)
    c_of = l // 32
    wib_of = (l % 32) // PATCH
    p2_of = l % PATCH
    idx = 48 * wib_of + 3 * p2_of + c_of
    fd = jnp.take_along_axis(fc, idx, axis=-1)

    planes = [
        jax.lax.slice(fd, (0, 0, 0, 32 * c), (TUBELET, 224, 7, 32 * (c + 1)))
        .reshape(TUBELET, 224, 224)
        for c in range(3)
    ]
    fr_ref[0] = jnp.stack(planes, axis=1)

    g = fd.reshape(TUBELET, 14, PATCH, 672)
    g = jnp.transpose(g, (1, 0, 2, 3))
    g = g.reshape(14, TUBELET * PATCH, 672)
    t = jnp.transpose(g, (0, 2, 1))
    t = t.reshape(14, 7, 3, TUBELET, PATCH, 32)
    t = jnp.transpose(t, (0, 1, 3, 2, 4, 5))
    t = jnp.transpose(t, (0, 1, 2, 3, 5, 4))
    tok_ref[...] = t.reshape(196, 3, TUBELET, PATCH, PATCH)


_POS_CHUNK = 56


def _pos_broadcast(B, total_tokens, pos_table):
    mesh = plsc.VectorSubcoreMesh(core_axis_name="c", subcore_axis_name="s")

    @functools.partial(
        pl.kernel,
        mesh=mesh,
        out_type=jax.ShapeDtypeStruct((B * total_tokens, EMBED_DIMS), jnp.float32),
        scratch_types=[
            pltpu.VMEM((_POS_CHUNK, EMBED_DIMS), jnp.float32),
        ],
    )
    def pos_kernel(tab_hbm, out_hbm, buf):
        wid = lax.axis_index("s") * 2 + lax.axis_index("c")
        nchunk = total_tokens // _POS_CHUNK

        @pl.when(wid < nchunk)
        def _():
            base = wid * _POS_CHUNK
            pltpu.sync_copy(tab_hbm.at[pl.ds(base, _POS_CHUNK)], buf)
            for b in range(B):
                pltpu.sync_copy(
                    buf, out_hbm.at[pl.ds(b * total_tokens + base, _POS_CHUNK)]
                )

    return pos_kernel(pos_table)


def kernel(frames, targets):
    B, T, H, W, C = frames.shape
    t = NUM_FRAMES // TUBELET
    h = H // PATCH
    w = W // PATCH
    total_tokens = t * h * w

    pos_table = jnp.asarray(_sinusoid_table(total_tokens, EMBED_DIMS))
    frames_wc = frames.reshape(B, T, H, W * C)

    grid = (B, t)
    tok, fr_t = pl.pallas_call(
        _body,
        grid=grid,
        in_specs=[
            pl.BlockSpec((1, TUBELET, H, W * C), lambda b, i: (b, i, 0, 0)),
        ],
        out_specs=[
            pl.BlockSpec(
                (h * w, C, TUBELET, PATCH, PATCH),
                lambda b, i: (b * 8 + i, 0, 0, 0, 0),
            ),
            pl.BlockSpec((1, TUBELET, C, H, W), lambda b, i: (b, i, 0, 0, 0)),
        ],
        out_shape=[
            jax.ShapeDtypeStruct(
                (B * total_tokens, C, TUBELET, PATCH, PATCH), frames.dtype
            ),
            jax.ShapeDtypeStruct((B, T, C, H, W), frames.dtype),
        ],
    )(frames_wc)

    pos_out = _pos_broadcast(B, total_tokens, pos_table)
    num_valid_tokens = jnp.full((B,), total_tokens, dtype=jnp.int32)
    token_mask = jnp.ones((B, total_tokens), dtype=bool)
    return (tok, num_valid_tokens, pos_out, token_mask, fr_t)


# speedup vs baseline: 1.9748x; 1.9748x over previous
import functools

import numpy as np
import jax
from jax import lax
import jax.numpy as jnp
from jax.experimental import pallas as pl
from jax.experimental.pallas import tpu as pltpu
from jax.experimental.pallas import tpu_sc as plsc

NUM_FRAMES = 16
TUBELET = 2
PATCH = 16
EMBED_DIMS = 768


def _sinusoid_table(n_position, embed_dims, base=10000):
    vec = np.arange(embed_dims, dtype=np.float64)
    vec = (vec - vec % 2) / embed_dims
    vec = np.power(float(base), -vec).reshape(1, -1)
    table = np.arange(n_position, dtype=np.float64).reshape(-1, 1) * vec
    table[:, 0::2] = np.sin(table[:, 0::2])
    table[:, 1::2] = np.cos(table[:, 1::2])
    return table.astype(np.float32)


def _body(x_ref, tok_ref, fr_ref):
    x = x_ref[0]
    fr_ref[0] = x

    vs = []
    for p0 in range(TUBELET):
        for c in range(3):
            plane = x[p0, c]
            pt = jnp.transpose(plane, (1, 0))
            v = pt.reshape(14, PATCH, 224)
            v = jnp.transpose(v, (0, 2, 1))
            v = v.reshape(14, 14, PATCH, PATCH)
            vs.append(jnp.transpose(v, (1, 0, 2, 3)))
    t = jnp.stack(vs, axis=0)
    t = t.reshape(TUBELET, 3, 14, 14, PATCH, PATCH)
    t = jnp.transpose(t, (2, 3, 1, 0, 4, 5))
    tok_ref[...] = t.reshape(196, 3, TUBELET, PATCH, PATCH)


_POS_CHUNK = 56


def _pos_broadcast(B, total_tokens, pos_table):
    mesh = plsc.VectorSubcoreMesh(core_axis_name="c", subcore_axis_name="s")

    @functools.partial(
        pl.kernel,
        mesh=mesh,
        out_type=jax.ShapeDtypeStruct((B * total_tokens, EMBED_DIMS), jnp.float32),
        scratch_types=[
            pltpu.VMEM((_POS_CHUNK, EMBED_DIMS), jnp.float32),
        ],
    )
    def pos_kernel(tab_hbm, out_hbm, buf):
        wid = lax.axis_index("s") * 2 + lax.axis_index("c")
        nchunk = total_tokens // _POS_CHUNK

        @pl.when(wid < nchunk)
        def _():
            base = wid * _POS_CHUNK
            pltpu.sync_copy(tab_hbm.at[pl.ds(base, _POS_CHUNK)], buf)
            for b in range(B):
                pltpu.sync_copy(
                    buf, out_hbm.at[pl.ds(b * total_tokens + base, _POS_CHUNK)]
                )

    return pos_kernel(pos_table)


def kernel(frames, targets):
    B, T, H, W, C = frames.shape
    t = NUM_FRAMES // TUBELET
    h = H // PATCH
    w = W // PATCH
    total_tokens = t * h * w

    pos_table = jnp.asarray(_sinusoid_table(total_tokens, EMBED_DIMS))
    xt = jnp.transpose(frames, (0, 1, 4, 2, 3))

    grid = (B, t)
    tok, fr_t = pl.pallas_call(
        _body,
        grid=grid,
        in_specs=[
            pl.BlockSpec((1, TUBELET, C, H, W), lambda b, i: (b, i, 0, 0, 0)),
        ],
        out_specs=[
            pl.BlockSpec(
                (h * w, C, TUBELET, PATCH, PATCH),
                lambda b, i: (b * 8 + i, 0, 0, 0, 0),
            ),
            pl.BlockSpec((1, TUBELET, C, H, W), lambda b, i: (b, i, 0, 0, 0)),
        ],
        out_shape=[
            jax.ShapeDtypeStruct(
                (B * total_tokens, C, TUBELET, PATCH, PATCH), frames.dtype
            ),
            jax.ShapeDtypeStruct((B, T, C, H, W), frames.dtype),
        ],
    )(xt)

    pos_out = _pos_broadcast(B, total_tokens, pos_table)
    num_valid_tokens = jnp.full((B,), total_tokens, dtype=jnp.int32)
    token_mask = jnp.ones((B, total_tokens), dtype=bool)
    return (tok, num_valid_tokens, pos_out, token_mask, fr_t)
